# Initial kernel scaffold; baseline (speedup 1.0000x reference)
import jax
import jax.numpy as jnp
from jax.experimental import pallas as pl


def kernel(l_x, l_edge_index, r_x, r_edge_index, Wl, att_src_l, att_dst_l, bl, Wr, att_src_r, att_dst_r, br, W1, b1, W2, b2, W3, b3):
    raise NotImplementedError("write your pallas kernel here")


# speedup vs baseline: 120.5881x; 120.5881x over previous
import jax
import jax.numpy as jnp
from jax import lax
from jax.experimental import pallas as pl
from jax.experimental.pallas import tpu as pltpu
from jax.experimental.pallas import tpu_sc as plsc

N = 10000
E = 320000
D = 128
NT = 16
EPT = E // NT
RPT = (EPT + 127) // 128
PAD_EPT = RPT * 128
M = 10240
SPT = M // NT

_HI = lax.Precision.HIGHEST


def _prologue_body(x_ref, w_ref, aat_ref, h_ref, o_ref, m_ref, mscr):
    i = pl.program_id(0)
    h_blk = jnp.dot(x_ref[...], w_ref[...])
    h_ref[...] = h_blk
    s1 = jnp.sum(h_blk * aat_ref[0:1, :], axis=1, keepdims=True)
    s2 = jnp.sum(h_blk * aat_ref[1:2, :], axis=1, keepdims=True)
    blk = jnp.concatenate([s1, s2], axis=1)
    o_ref[...] = blk
    bm = jnp.max(blk, axis=0, keepdims=True)

    @pl.when(i == 0)
    def _():
        mscr[...] = bm

    @pl.when(i > 0)
    def _():
        mscr[...] = jnp.maximum(mscr[...], bm)

    @pl.when(i == pl.num_programs(0) - 1)
    def _():
        m_ref[...] = mscr[...]


def _prologue(x, w, aat):
    bn = 1000
    return pl.pallas_call(
        _prologue_body,
        grid=(N // bn,),
        in_specs=[
            pl.BlockSpec((bn, D), lambda i: (i, 0)),
            pl.BlockSpec((D, D), lambda i: (0, 0)),
            pl.BlockSpec((2, D), lambda i: (0, 0)),
        ],
        out_specs=[
            pl.BlockSpec((bn, D), lambda i: (i, 0)),
            pl.BlockSpec((bn, 2), lambda i: (i, 0)),
            pl.BlockSpec((1, 2), lambda i: (0, 0)),
        ],
        out_shape=[
            jax.ShapeDtypeStruct((N, D), jnp.float32),
            jax.ShapeDtypeStruct((N, 2), jnp.float32),
            jax.ShapeDtypeStruct((1, 2), jnp.float32),
        ],
        scratch_shapes=[pltpu.VMEM((1, 2), jnp.float32)],
    )(x, w, aat)


def _sc_body(tabs, eidx, mvec, c_out,
             as_t, ad_t, src_v, dst_v, ee_v, den_v, c_v, m_v,
             dself_v, red_v, tmp_v, sh_part, sh_red):
    g = lax.axis_index("c")
    s = lax.axis_index("s")

    pltpu.sync_copy(tabs.at[g, 0], as_t)
    pltpu.sync_copy(tabs.at[g, 1], ad_t)
    pltpu.sync_copy(eidx.at[g, 0, s], src_v)
    pltpu.sync_copy(eidx.at[g, 1, s], dst_v)
    pltpu.sync_copy(mvec.at[g], m_v)

    zero = jnp.zeros((16,), jnp.float32)

    def zr(i, _):
        den_v[pl.ds(i * 16, 16)] = zero
        c_v[pl.ds(i * 16, 16)] = zero
        return 0
    lax.fori_loop(0, M // 16, zr, 0)

    m_s = m_v[...]

    def row1(r, _):
        def sub1(k, _):
            i_s = src_v[r, pl.ds(k * 16, 16)]
            i_d = dst_v[r, pl.ds(k * 16, 16)]
            a_s = plsc.load_gather(as_t, [i_s])
            a_d = plsc.load_gather(ad_t, [i_d])
            sv = a_s + a_d
            e = jnp.maximum(sv, sv * 0.2)
            eev = jnp.exp(e - m_s)
            ee_v[r, pl.ds(k * 16, 16)] = eev
            plsc.addupdate_scatter(den_v, [i_d], eev)
            return 0
        lax.fori_loop(0, 8, sub1, 0)
        return 0
    lax.fori_loop(0, RPT, row1, 0)

    def selfd(j, _):
        off = s * SPT + j * 16
        sv = as_t[pl.ds(off, 16)] + ad_t[pl.ds(off, 16)]
        dself = jnp.exp(jnp.maximum(sv, sv * 0.2) - m_s)
        dself_v[pl.ds(j * 16, 16)] = dself
        den_v[pl.ds(off, 16)] = den_v[pl.ds(off, 16)] + dself
        return 0
    lax.fori_loop(0, SPT // 16, selfd, 0)

    pltpu.sync_copy(den_v, sh_part.at[s])
    plsc.subcore_barrier()
    pltpu.sync_copy(sh_part.at[0, pl.ds(s * SPT, SPT)], red_v)

    def acc(k, _):
        pltpu.sync_copy(sh_part.at[k, pl.ds(s * SPT, SPT)], tmp_v)

        def av(j, _):
            red_v[pl.ds(j * 16, 16)] = (red_v[pl.ds(j * 16, 16)] +
                                        tmp_v[pl.ds(j * 16, 16)])
            return 0
        lax.fori_loop(0, SPT // 16, av, 0)
        return 0
    lax.fori_loop(1, NT, acc, 0)
    pltpu.sync_copy(red_v, sh_red.at[pl.ds(s * SPT, SPT)])
    plsc.subcore_barrier()
    pltpu.sync_copy(sh_red, den_v)

    def selfc(j, _):
        off = s * SPT + j * 16
        dtot = den_v[pl.ds(off, 16)]
        c_v[pl.ds(off, 16)] = dself_v[pl.ds(j * 16, 16)] / (dtot + 1e-16)
        return 0
    lax.fori_loop(0, SPT // 16, selfc, 0)

    def row2(r, _):
        def sub2(k, _):
            i_s = src_v[r, pl.ds(k * 16, 16)]
            i_d = dst_v[r, pl.ds(k * 16, 16)]
            eev = ee_v[r, pl.ds(k * 16, 16)]
            dv = plsc.load_gather(den_v, [i_d])
            al = eev / (dv + 1e-16)
            plsc.addupdate_scatter(c_v, [i_s], al)
            return 0
        lax.fori_loop(0, 8, sub2, 0)
        return 0
    lax.fori_loop(0, RPT, row2, 0)

    pltpu.sync_copy(c_v, sh_part.at[s])
    plsc.subcore_barrier()
    pltpu.sync_copy(sh_part.at[0, pl.ds(s * SPT, SPT)], red_v)
    lax.fori_loop(1, NT, acc, 0)
    pltpu.sync_copy(red_v, c_out.at[g, pl.ds(s * SPT, SPT)])


def _sc_kernel(tabs, eidx, mvec):
    k = pl.kernel(
        _sc_body,
        out_type=jax.ShapeDtypeStruct((2, M), jnp.float32),
        mesh=plsc.VectorSubcoreMesh(core_axis_name="c", subcore_axis_name="s"),
        compiler_params=pltpu.CompilerParams(needs_layout_passes=False),
        scratch_types=[
            pltpu.VMEM((M,), jnp.float32),
            pltpu.VMEM((M,), jnp.float32),
            pltpu.VMEM((RPT, 128), jnp.int32),
            pltpu.VMEM((RPT, 128), jnp.int32),
            pltpu.VMEM((RPT, 128), jnp.float32),
            pltpu.VMEM((M,), jnp.float32),
            pltpu.VMEM((M,), jnp.float32),
            pltpu.VMEM((16,), jnp.float32),
            pltpu.VMEM((SPT,), jnp.float32),
            pltpu.VMEM((SPT,), jnp.float32),
            pltpu.VMEM((SPT,), jnp.float32),
            pltpu.VMEM_SHARED((NT, M), jnp.float32),
            pltpu.VMEM_SHARED((M,), jnp.float32),
        ],
    )
    return k(tabs, eidx, mvec)


def _feat_body(c2t_ref, hl_ref, hr_ref, bvec_ref, feat_ref, cxl_s, cxr_s):
    i = pl.program_id(0)
    dn = (((0,), (0,)), ((), ()))
    pl_ = lax.dot_general(c2t_ref[:, 0:1], hl_ref[...], dn,
                          precision=_HI, preferred_element_type=jnp.float32)
    pr_ = lax.dot_general(c2t_ref[:, 1:2], hr_ref[...], dn,
                          precision=_HI, preferred_element_type=jnp.float32)

    @pl.when(i == 0)
    def _():
        cxl_s[...] = pl_
        cxr_s[...] = pr_

    @pl.when(i > 0)
    def _():
        cxl_s[...] = cxl_s[...] + pl_
        cxr_s[...] = cxr_s[...] + pr_

    @pl.when(i == pl.num_programs(0) - 1)
    def _():
        suma = cxl_s[...] + float(N) * bvec_ref[0:1, :]
        sumb = cxr_s[...] + float(N) * bvec_ref[1:2, :]
        feat_ref[...] = jnp.concatenate([suma, sumb], axis=1)


def _feat(c2t, hl, hr, bvec):
    bn = 1000
    return pl.pallas_call(
        _feat_body,
        grid=(N // bn,),
        in_specs=[
            pl.BlockSpec((bn, 2), lambda i: (i, 0)),
            pl.BlockSpec((bn, D), lambda i: (i, 0)),
            pl.BlockSpec((bn, D), lambda i: (i, 0)),
            pl.BlockSpec((2, D), lambda i: (0, 0)),
        ],
        out_specs=pl.BlockSpec((1, 2 * D), lambda i: (0, 0)),
        out_shape=jax.ShapeDtypeStruct((1, 2 * D), jnp.float32),
        scratch_shapes=[pltpu.VMEM((1, D), jnp.float32),
                        pltpu.VMEM((1, D), jnp.float32)],
    )(c2t, hl, hr, bvec)


def _head_body(feat_ref, w1_ref, b1_ref, w2_ref, w3_ref, b2_ref, b3_ref,
               out_ref, h2_s):
    k = pl.program_id(0)
    h1_k = jnp.dot(feat_ref[...], w1_ref[...]) + b1_ref[...]
    p2 = jnp.dot(h1_k, w2_ref[...])

    @pl.when(k == 0)
    def _():
        h2_s[...] = p2

    @pl.when(k > 0)
    def _():
        h2_s[...] = h2_s[...] + p2

    @pl.when(k == pl.num_programs(0) - 1)
    def _():
        h2 = h2_s[...] + b2_ref[...]
        out_ref[...] = jnp.dot(h2, w3_ref[...]) + b3_ref[...]


def _head(feat, w1, b1r, w2, w3, b2r, b3r):
    bk = 2048
    kk = w1.shape[1] // bk
    return pl.pallas_call(
        _head_body,
        grid=(kk,),
        in_specs=[
            pl.BlockSpec((1, 2 * D), lambda k: (0, 0)),
            pl.BlockSpec((2 * D, bk), lambda k: (0, k)),
            pl.BlockSpec((1, bk), lambda k: (0, k)),
            pl.BlockSpec((bk, D), lambda k: (k, 0)),
            pl.BlockSpec((D, 1), lambda k: (0, 0)),
            pl.BlockSpec((1, D), lambda k: (0, 0)),
            pl.BlockSpec((1, 1), lambda k: (0, 0)),
        ],
        out_specs=pl.BlockSpec((1, 1), lambda k: (0, 0)),
        out_shape=jax.ShapeDtypeStruct((1, 1), jnp.float32),
        scratch_shapes=[pltpu.VMEM((1, D), jnp.float32)],
    )(feat, w1, b1r, w2, w3, b2r, b3r)


def _prep_edges(ei):
    pad_idx = (N + (jnp.arange(PAD_EPT - EPT) % (M - N))).astype(jnp.int32)
    s = ei.reshape(2, NT, EPT)
    padb = jnp.broadcast_to(pad_idx, (2, NT, PAD_EPT - EPT))
    return jnp.concatenate([s, padb], axis=2).reshape(2, NT, RPT, 128)


def kernel(l_x, l_edge_index, r_x, r_edge_index, Wl, att_src_l, att_dst_l, bl,
           Wr, att_src_r, att_dst_r, br, W1, b1, W2, b2, W3, b3):
    aat_l = jnp.stack([att_src_l, att_dst_l])
    aat_r = jnp.stack([att_src_r, att_dst_r])
    h_l, asad_l, mm_l = _prologue(l_x, Wl, aat_l)
    h_r, asad_r, mm_r = _prologue(r_x, Wr, aat_r)

    tabs = jnp.stack([
        jnp.pad(asad_l, ((0, M - N), (0, 0))).T,
        jnp.pad(asad_r, ((0, M - N), (0, 0))).T,
    ])
    m_l = jnp.maximum(mm_l[0, 0] + mm_l[0, 1], 0.0)
    m_r = jnp.maximum(mm_r[0, 0] + mm_r[0, 1], 0.0)
    mvec = jnp.broadcast_to(jnp.stack([m_l, m_r])[:, None], (2, 16))
    mvec = jnp.asarray(mvec, jnp.float32)

    eidx = jnp.stack([_prep_edges(l_edge_index), _prep_edges(r_edge_index)])

    c2 = _sc_kernel(tabs, eidx, mvec)

    bvec = jnp.stack([bl, br])
    feat = _feat(c2.T, h_l, h_r, bvec)
    out = _head(feat, W1, b1.reshape(1, -1), W2, W3,
                b2.reshape(1, -1), b3.reshape(1, 1))
    return out.reshape(1)


# speedup vs baseline: 196.2036x; 1.6271x over previous
import jax
import jax.numpy as jnp
from jax import lax
from jax.experimental import pallas as pl
from jax.experimental.pallas import tpu as pltpu
from jax.experimental.pallas import tpu_sc as plsc

N = 10000
E = 320000
D = 128
NT = 16
EPT = E // NT
VPT = EPT // 16
M = 10240
SPT = M // NT

_HI = lax.Precision.HIGHEST


def _prologue_body(x_ref, w_ref, aat_ref, h_ref, o_ref, m_ref, mscr):
    i = pl.program_id(0)
    h_blk = jnp.dot(x_ref[...], w_ref[...])
    h_ref[...] = h_blk
    s1 = jnp.sum(h_blk * aat_ref[0:1, :], axis=1, keepdims=True)
    s2 = jnp.sum(h_blk * aat_ref[1:2, :], axis=1, keepdims=True)
    blk = jnp.concatenate([s1, s2], axis=1)
    o_ref[...] = blk
    bm = jnp.max(blk, axis=0, keepdims=True)

    @pl.when(i == 0)
    def _():
        mscr[...] = bm

    @pl.when(i > 0)
    def _():
        mscr[...] = jnp.maximum(mscr[...], bm)

    @pl.when(i == pl.num_programs(0) - 1)
    def _():
        m_ref[...] = mscr[...]


def _prologue(x, w, aat):
    bn = 2000
    return pl.pallas_call(
        _prologue_body,
        grid=(N // bn,),
        in_specs=[
            pl.BlockSpec((bn, D), lambda i: (i, 0)),
            pl.BlockSpec((D, D), lambda i: (0, 0)),
            pl.BlockSpec((2, D), lambda i: (0, 0)),
        ],
        out_specs=[
            pl.BlockSpec((bn, D), lambda i: (i, 0)),
            pl.BlockSpec((bn, 2), lambda i: (i, 0)),
            pl.BlockSpec((1, 2), lambda i: (0, 0)),
        ],
        out_shape=[
            jax.ShapeDtypeStruct((N, D), jnp.float32),
            jax.ShapeDtypeStruct((N, 2), jnp.float32),
            jax.ShapeDtypeStruct((1, 2), jnp.float32),
        ],
        scratch_shapes=[pltpu.VMEM((1, 2), jnp.float32)],
    )(x, w, aat)


def _sc_body(tabs, lei, rei, mvec, c_out,
             as_t, ad_t, src_v, dst_v, ee_v, den_v, c_v, m_v,
             dself_v, red_v, big_v, sh_part, sh_red):
    g = lax.axis_index("c")
    s = lax.axis_index("s")

    pltpu.sync_copy(tabs.at[g, 0], as_t)
    pltpu.sync_copy(tabs.at[g, 1], ad_t)
    pltpu.sync_copy(mvec.at[g], m_v)

    @pl.when(g == 0)
    def _():
        pltpu.sync_copy(lei.at[pl.ds(s * EPT, EPT)], src_v)
        pltpu.sync_copy(lei.at[pl.ds(E + s * EPT, EPT)], dst_v)

    @pl.when(g == 1)
    def _():
        pltpu.sync_copy(rei.at[pl.ds(s * EPT, EPT)], src_v)
        pltpu.sync_copy(rei.at[pl.ds(E + s * EPT, EPT)], dst_v)

    zero = jnp.zeros((16,), jnp.float32)

    @plsc.parallel_loop(0, M // 16, unroll=8)
    def _(i):
        den_v[pl.ds(i * 16, 16)] = zero
        c_v[pl.ds(i * 16, 16)] = zero

    m_s = m_v[...]

    @plsc.parallel_loop(0, VPT, unroll=8)
    def _(i):
        i_s = src_v[pl.ds(i * 16, 16)]
        i_d = dst_v[pl.ds(i * 16, 16)]
        a_s = plsc.load_gather(as_t, [i_s])
        a_d = plsc.load_gather(ad_t, [i_d])
        sv = a_s + a_d
        e = jnp.maximum(sv, sv * 0.2)
        eev = jnp.exp(e - m_s)
        ee_v[pl.ds(i * 16, 16)] = eev
        plsc.addupdate_scatter(den_v, [i_d], eev)

    @plsc.parallel_loop(0, SPT // 16, unroll=4)
    def _(j):
        off = s * SPT + j * 16
        sv = as_t[pl.ds(off, 16)] + ad_t[pl.ds(off, 16)]
        dself = jnp.exp(jnp.maximum(sv, sv * 0.2) - m_s)
        dself_v[pl.ds(j * 16, 16)] = dself
        den_v[pl.ds(off, 16)] = den_v[pl.ds(off, 16)] + dself

    pltpu.sync_copy(den_v, sh_part.at[s])
    plsc.subcore_barrier()
    pltpu.sync_copy(sh_part.at[:, pl.ds(s * SPT, SPT)], big_v)

    def _reduce_cols():
        @plsc.parallel_loop(0, SPT // 16, unroll=4)
        def _(j):
            acc = big_v[0, pl.ds(j * 16, 16)]
            for r in range(1, NT):
                acc = acc + big_v[r, pl.ds(j * 16, 16)]
            red_v[pl.ds(j * 16, 16)] = acc

    _reduce_cols()
    pltpu.sync_copy(red_v, sh_red.at[pl.ds(s * SPT, SPT)])
    plsc.subcore_barrier()
    pltpu.sync_copy(sh_red, den_v)

    @plsc.parallel_loop(0, SPT // 16, unroll=4)
    def _(j):
        off = s * SPT + j * 16
        dtot = den_v[pl.ds(off, 16)]
        c_v[pl.ds(off, 16)] = dself_v[pl.ds(j * 16, 16)] / (dtot + 1e-16)

    @plsc.parallel_loop(0, VPT, unroll=8)
    def _(i):
        i_s = src_v[pl.ds(i * 16, 16)]
        i_d = dst_v[pl.ds(i * 16, 16)]
        eev = ee_v[pl.ds(i * 16, 16)]
        dv = plsc.load_gather(den_v, [i_d])
        al = eev / (dv + 1e-16)
        plsc.addupdate_scatter(c_v, [i_s], al)

    pltpu.sync_copy(c_v, sh_part.at[s])
    plsc.subcore_barrier()
    pltpu.sync_copy(sh_part.at[:, pl.ds(s * SPT, SPT)], big_v)
    _reduce_cols()
    pltpu.sync_copy(red_v, c_out.at[g, pl.ds(s * SPT, SPT)])


def _sc_kernel(tabs, lei, rei, mvec):
    k = pl.kernel(
        _sc_body,
        out_type=jax.ShapeDtypeStruct((2, M), jnp.float32),
        mesh=plsc.VectorSubcoreMesh(core_axis_name="c", subcore_axis_name="s"),
        compiler_params=pltpu.CompilerParams(needs_layout_passes=False),
        scratch_types=[
            pltpu.VMEM((M,), jnp.float32),
            pltpu.VMEM((M,), jnp.float32),
            pltpu.VMEM((EPT,), jnp.int32),
            pltpu.VMEM((EPT,), jnp.int32),
            pltpu.VMEM((EPT,), jnp.float32),
            pltpu.VMEM((M,), jnp.float32),
            pltpu.VMEM((M,), jnp.float32),
            pltpu.VMEM((16,), jnp.float32),
            pltpu.VMEM((SPT,), jnp.float32),
            pltpu.VMEM((SPT,), jnp.float32),
            pltpu.VMEM((NT, SPT), jnp.float32),
            pltpu.VMEM_SHARED((NT, M), jnp.float32),
            pltpu.VMEM_SHARED((M,), jnp.float32),
        ],
    )
    return k(tabs, lei, rei, mvec)


def _tail_body(c2t_ref, hl_ref, hr_ref, bvec_ref, w1_ref, b1_ref, w2_ref,
               w3_ref, b2_ref, b3_ref, out_ref, cxl_s, cxr_s, feat_s, h2_s):
    i = pl.program_id(0)
    nblk = N // 1000

    @pl.when(i < nblk)
    def _():
        dn = (((0,), (0,)), ((), ()))
        pl_ = lax.dot_general(c2t_ref[:, 0:1], hl_ref[...], dn,
                              precision=_HI,
                              preferred_element_type=jnp.float32)
        pr_ = lax.dot_general(c2t_ref[:, 1:2], hr_ref[...], dn,
                              precision=_HI,
                              preferred_element_type=jnp.float32)

        @pl.when(i == 0)
        def _():
            cxl_s[...] = pl_
            cxr_s[...] = pr_

        @pl.when(i > 0)
        def _():
            cxl_s[...] = cxl_s[...] + pl_
            cxr_s[...] = cxr_s[...] + pr_

        @pl.when(i == nblk - 1)
        def _():
            suma = cxl_s[...] + float(N) * bvec_ref[0:1, :]
            sumb = cxr_s[...] + float(N) * bvec_ref[1:2, :]
            feat_s[...] = jnp.concatenate([suma, sumb], axis=1)

    @pl.when(i >= nblk)
    def _():
        h1_k = jnp.dot(feat_s[...], w1_ref[...]) + b1_ref[...]
        p2 = jnp.dot(h1_k, w2_ref[...])

        @pl.when(i == nblk)
        def _():
            h2_s[...] = p2

        @pl.when(i > nblk)
        def _():
            h2_s[...] = h2_s[...] + p2

        @pl.when(i == pl.num_programs(0) - 1)
        def _():
            h2 = h2_s[...] + b2_ref[...]
            out_ref[...] = jnp.dot(h2, w3_ref[...]) + b3_ref[...]


def _tail(c2t, hl, hr, bvec, w1, b1r, w2, w3, b2r, b3r):
    bn = 1000
    bk = 2048
    nblk = N // bn
    kk = w1.shape[1] // bk

    def fid(i):
        return jnp.minimum(i, nblk - 1)

    def kid(i):
        return jnp.clip(i - nblk, 0, kk - 1)

    return pl.pallas_call(
        _tail_body,
        grid=(nblk + kk,),
        in_specs=[
            pl.BlockSpec((bn, 2), lambda i: (fid(i), 0)),
            pl.BlockSpec((bn, D), lambda i: (fid(i), 0)),
            pl.BlockSpec((bn, D), lambda i: (fid(i), 0)),
            pl.BlockSpec((2, D), lambda i: (0, 0)),
            pl.BlockSpec((2 * D, bk), lambda i: (0, kid(i))),
            pl.BlockSpec((1, bk), lambda i: (0, kid(i))),
            pl.BlockSpec((bk, D), lambda i: (kid(i), 0)),
            pl.BlockSpec((D, 1), lambda i: (0, 0)),
            pl.BlockSpec((1, D), lambda i: (0, 0)),
            pl.BlockSpec((1, 1), lambda i: (0, 0)),
        ],
        out_specs=pl.BlockSpec((1, 1), lambda i: (0, 0)),
        out_shape=jax.ShapeDtypeStruct((1, 1), jnp.float32),
        scratch_shapes=[pltpu.VMEM((1, D), jnp.float32),
                        pltpu.VMEM((1, D), jnp.float32),
                        pltpu.VMEM((1, 2 * D), jnp.float32),
                        pltpu.VMEM((1, D), jnp.float32)],
    )(c2t, hl, hr, bvec, w1, b1r, w2, w3, b2r, b3r)


def kernel(l_x, l_edge_index, r_x, r_edge_index, Wl, att_src_l, att_dst_l, bl,
           Wr, att_src_r, att_dst_r, br, W1, b1, W2, b2, W3, b3):
    aat_l = jnp.stack([att_src_l, att_dst_l])
    aat_r = jnp.stack([att_src_r, att_dst_r])
    h_l, asad_l, mm_l = _prologue(l_x, Wl, aat_l)
    h_r, asad_r, mm_r = _prologue(r_x, Wr, aat_r)

    tabs = jnp.stack([
        jnp.pad(asad_l, ((0, M - N), (0, 0))).T,
        jnp.pad(asad_r, ((0, M - N), (0, 0))).T,
    ])
    m_l = jnp.maximum(mm_l[0, 0] + mm_l[0, 1], 0.0)
    m_r = jnp.maximum(mm_r[0, 0] + mm_r[0, 1], 0.0)
    mvec = jnp.broadcast_to(jnp.stack([m_l, m_r])[:, None], (2, 16))
    mvec = jnp.asarray(mvec, jnp.float32)

    c2 = _sc_kernel(tabs, l_edge_index.reshape(-1),
                    r_edge_index.reshape(-1), mvec)

    bvec = jnp.stack([bl, br])
    out = _tail(c2.T, h_l, h_r, bvec, W1, b1.reshape(1, -1), W2, W3,
                b2.reshape(1, -1), b3.reshape(1, 1))
    return out.reshape(1)


# speedup vs baseline: 236.9154x; 1.2075x over previous
import jax
import jax.numpy as jnp
from jax import lax
from jax.experimental import pallas as pl
from jax.experimental.pallas import tpu as pltpu
from jax.experimental.pallas import tpu_sc as plsc

N = 10000
E = 320000
D = 128
NT = 16
EPT = E // NT
VPT = EPT // 16
M = 10240
SPT = M // NT

_HI = lax.Precision.HIGHEST


def _prologue_body(x_ref, w_ref, aat_ref, o_ref, m_ref, mscr):
    i = pl.program_id(0)
    h_blk = jnp.dot(x_ref[...], w_ref[...])
    s1 = jnp.sum(h_blk * aat_ref[0:1, :], axis=1, keepdims=True)
    s2 = jnp.sum(h_blk * aat_ref[1:2, :], axis=1, keepdims=True)
    blk = jnp.concatenate([s1, s2], axis=1)
    o_ref[...] = blk
    bm = jnp.max(blk, axis=0, keepdims=True)

    @pl.when(i == 0)
    def _():
        mscr[...] = bm

    @pl.when(i > 0)
    def _():
        mscr[...] = jnp.maximum(mscr[...], bm)

    @pl.when(i == pl.num_programs(0) - 1)
    def _():
        m_ref[...] = mscr[...]


def _prologue(x, w, aat):
    bn = 2000
    return pl.pallas_call(
        _prologue_body,
        grid=(N // bn,),
        in_specs=[
            pl.BlockSpec((bn, D), lambda i: (i, 0)),
            pl.BlockSpec((D, D), lambda i: (0, 0)),
            pl.BlockSpec((2, D), lambda i: (0, 0)),
        ],
        out_specs=[
            pl.BlockSpec((bn, 2), lambda i: (i, 0)),
            pl.BlockSpec((1, 2), lambda i: (0, 0)),
        ],
        out_shape=[
            jax.ShapeDtypeStruct((N, 2), jnp.float32),
            jax.ShapeDtypeStruct((1, 2), jnp.float32),
        ],
        scratch_shapes=[pltpu.VMEM((1, 2), jnp.float32)],
    )(x, w, aat)


_BIG = 157 * 128
_SML = 156 * 128


def _sc_body(tabs, lei, rei, mvec, c_out,
             as_t, ad_t, ei_v, ee_v, den_v, c_v, m_v,
             dself_v, red_v, big_v, dma_sem, sh_part, sh_red):
    g = lax.axis_index("c")
    s = lax.axis_index("s")
    e_off = 128 * (156 * s + jnp.minimum(s, 4))

    for gv, ei in ((0, lei), (1, rei)):
        @pl.when((g == gv) & (s < 4))
        def _():
            pltpu.async_copy(ei.at[:, pl.ds(e_off, _BIG)], ei_v, dma_sem)

        @pl.when((g == gv) & (s >= 4))
        def _():
            pltpu.async_copy(ei.at[:, pl.ds(e_off, _SML)],
                             ei_v.at[:, pl.ds(0, _SML)], dma_sem)

    pltpu.sync_copy(tabs.at[g, 0], as_t)
    pltpu.sync_copy(tabs.at[g, 1], ad_t)
    pltpu.sync_copy(mvec.at[g], m_v)

    zero = jnp.zeros((16,), jnp.float32)

    @plsc.parallel_loop(0, M // 16, unroll=8)
    def _(i):
        den_v[pl.ds(i * 16, 16)] = zero
        c_v[pl.ds(i * 16, 16)] = zero

    @pl.when(s < 4)
    def _():
        pltpu.make_async_copy(lei.at[:, pl.ds(0, _BIG)], ei_v, dma_sem).wait()

    @pl.when(s >= 4)
    def _():
        pltpu.make_async_copy(lei.at[:, pl.ds(0, _SML)],
                              ei_v.at[:, pl.ds(0, _SML)], dma_sem).wait()

    m_s = m_v[...]

    def pass1(nv):
        @plsc.parallel_loop(0, nv, unroll=8)
        def _(i):
            i_s = ei_v[0, pl.ds(i * 16, 16)]
            i_d = ei_v[1, pl.ds(i * 16, 16)]
            a_s = plsc.load_gather(as_t, [i_s])
            a_d = plsc.load_gather(ad_t, [i_d])
            sv = a_s + a_d
            e = jnp.maximum(sv, sv * 0.2)
            ub = jnp.maximum(a_d + m_s, 0.0)
            eev = jnp.exp(e - ub)
            ee_v[pl.ds(i * 16, 16)] = eev
            plsc.addupdate_scatter(den_v, [i_d], eev)

    @pl.when(s < 4)
    def _():
        pass1(_BIG // 16)

    @pl.when(s >= 4)
    def _():
        pass1(_SML // 16)

    @plsc.parallel_loop(0, SPT // 16, unroll=4)
    def _(j):
        off = s * SPT + j * 16
        a_d = ad_t[pl.ds(off, 16)]
        sv = as_t[pl.ds(off, 16)] + a_d
        ub = jnp.maximum(a_d + m_s, 0.0)
        dself = jnp.exp(jnp.maximum(sv, sv * 0.2) - ub)
        dself_v[pl.ds(j * 16, 16)] = dself
        den_v[pl.ds(off, 16)] = den_v[pl.ds(off, 16)] + dself

    pltpu.sync_copy(den_v, sh_part.at[s])
    plsc.subcore_barrier()
    pltpu.sync_copy(sh_part.at[:, pl.ds(s * SPT, SPT)], big_v)

    def _reduce_cols():
        @plsc.parallel_loop(0, SPT // 16, unroll=4)
        def _(j):
            acc = big_v[0, pl.ds(j * 16, 16)]
            for r in range(1, NT):
                acc = acc + big_v[r, pl.ds(j * 16, 16)]
            red_v[pl.ds(j * 16, 16)] = acc

    _reduce_cols()
    pltpu.sync_copy(red_v, sh_red.at[pl.ds(s * SPT, SPT)])
    plsc.subcore_barrier()
    pltpu.sync_copy(sh_red, den_v)

    @plsc.parallel_loop(0, SPT // 16, unroll=4)
    def _(j):
        off = s * SPT + j * 16
        dtot = den_v[pl.ds(off, 16)]
        c_v[pl.ds(off, 16)] = dself_v[pl.ds(j * 16, 16)] / (dtot + 1e-16)

    def pass2(nv):
        @plsc.parallel_loop(0, nv, unroll=8)
        def _(i):
            i_s = ei_v[0, pl.ds(i * 16, 16)]
            i_d = ei_v[1, pl.ds(i * 16, 16)]
            eev = ee_v[pl.ds(i * 16, 16)]
            dv = plsc.load_gather(den_v, [i_d])
            al = eev / (dv + 1e-16)
            plsc.addupdate_scatter(c_v, [i_s], al)

    @pl.when(s < 4)
    def _():
        pass2(_BIG // 16)

    @pl.when(s >= 4)
    def _():
        pass2(_SML // 16)

    pltpu.sync_copy(c_v, sh_part.at[s])
    plsc.subcore_barrier()
    pltpu.sync_copy(sh_part.at[:, pl.ds(s * SPT, SPT)], big_v)
    _reduce_cols()
    pltpu.sync_copy(red_v, c_out.at[g, pl.ds(s * SPT, SPT)])


def _sc_kernel(tabs, lei, rei, mvec):
    k = pl.kernel(
        _sc_body,
        out_type=jax.ShapeDtypeStruct((2, M), jnp.float32),
        mesh=plsc.VectorSubcoreMesh(core_axis_name="c", subcore_axis_name="s"),
        compiler_params=pltpu.CompilerParams(needs_layout_passes=False),
        scratch_types=[
            pltpu.VMEM((M,), jnp.float32),
            pltpu.VMEM((M,), jnp.float32),
            pltpu.VMEM((2, _BIG), jnp.int32),
            pltpu.VMEM((_BIG,), jnp.float32),
            pltpu.VMEM((M,), jnp.float32),
            pltpu.VMEM((M,), jnp.float32),
            pltpu.VMEM((16,), jnp.float32),
            pltpu.VMEM((SPT,), jnp.float32),
            pltpu.VMEM((SPT,), jnp.float32),
            pltpu.VMEM((NT, SPT), jnp.float32),
            pltpu.SemaphoreType.DMA,
            pltpu.VMEM_SHARED((NT, M), jnp.float32),
            pltpu.VMEM_SHARED((M,), jnp.float32),
        ],
    )
    return k(tabs, lei, rei, mvec)


def _tail_body(c2t_ref, xl_ref, xr_ref, wl_ref, wr_ref, bvec_ref, w1_ref,
               b1_ref, w2_ref, w3_ref, b2_ref, b3_ref, out_ref,
               cxl_s, cxr_s, feat_s):
    i = pl.program_id(0)
    nblk = N // 1000

    @pl.when(i < nblk)
    def _():
        hl_m = jnp.dot(xl_ref[...], wl_ref[...])
        hr_m = jnp.dot(xr_ref[...], wr_ref[...])
        pl_ = jnp.sum(hl_m * c2t_ref[:, 0:1], axis=0, keepdims=True)
        pr_ = jnp.sum(hr_m * c2t_ref[:, 1:2], axis=0, keepdims=True)

        @pl.when(i == 0)
        def _():
            cxl_s[...] = pl_
            cxr_s[...] = pr_

        @pl.when(i > 0)
        def _():
            cxl_s[...] = cxl_s[...] + pl_
            cxr_s[...] = cxr_s[...] + pr_

        @pl.when(i == nblk - 1)
        def _():
            suma = cxl_s[...] + float(N) * bvec_ref[0:1, :]
            sumb = cxr_s[...] + float(N) * bvec_ref[1:2, :]
            feat_s[...] = jnp.concatenate([suma, sumb], axis=1)

    @pl.when(i == nblk)
    def _():
        h1 = jnp.dot(feat_s[...], w1_ref[...]) + b1_ref[...]
        h2 = jnp.dot(h1, w2_ref[...]) + b2_ref[...]
        out_ref[...] = jnp.dot(h2, w3_ref[...]) + b3_ref[...]


def _tail(c2t, xl, xr, wl, wr, bvec, w1, b1r, w2, w3, b2r, b3r):
    bn = 1000
    nblk = N // bn
    kh = w1.shape[1]

    def fid(i):
        return jnp.minimum(i, nblk - 1)

    return pl.pallas_call(
        _tail_body,
        grid=(nblk + 1,),
        in_specs=[
            pl.BlockSpec((bn, 2), lambda i: (fid(i), 0)),
            pl.BlockSpec((bn, D), lambda i: (fid(i), 0)),
            pl.BlockSpec((bn, D), lambda i: (fid(i), 0)),
            pl.BlockSpec((D, D), lambda i: (0, 0)),
            pl.BlockSpec((D, D), lambda i: (0, 0)),
            pl.BlockSpec((2, D), lambda i: (0, 0)),
            pl.BlockSpec((2 * D, kh), lambda i: (0, 0)),
            pl.BlockSpec((1, kh), lambda i: (0, 0)),
            pl.BlockSpec((kh, D), lambda i: (0, 0)),
            pl.BlockSpec((D, 1), lambda i: (0, 0)),
            pl.BlockSpec((1, D), lambda i: (0, 0)),
            pl.BlockSpec((1, 1), lambda i: (0, 0)),
        ],
        out_specs=pl.BlockSpec((1, 1), lambda i: (0, 0)),
        out_shape=jax.ShapeDtypeStruct((1, 1), jnp.float32),
        scratch_shapes=[pltpu.VMEM((1, D), jnp.float32),
                        pltpu.VMEM((1, D), jnp.float32),
                        pltpu.VMEM((1, 2 * D), jnp.float32)],
    )(c2t, xl, xr, wl, wr, bvec, w1, b1r, w2, w3, b2r, b3r)


def kernel(l_x, l_edge_index, r_x, r_edge_index, Wl, att_src_l, att_dst_l, bl,
           Wr, att_src_r, att_dst_r, br, W1, b1, W2, b2, W3, b3):
    aat_l = jnp.stack([att_src_l, att_dst_l])
    aat_r = jnp.stack([att_src_r, att_dst_r])
    asad_l, mm_l = _prologue(l_x, Wl, aat_l)
    asad_r, mm_r = _prologue(r_x, Wr, aat_r)

    tabs = jnp.stack([
        jnp.pad(asad_l, ((0, M - N), (0, 0))).T,
        jnp.pad(asad_r, ((0, M - N), (0, 0))).T,
    ])
    mvec = jnp.broadcast_to(jnp.stack([mm_l[0, 0], mm_r[0, 0]])[:, None],
                            (2, 16))
    mvec = jnp.asarray(mvec, jnp.float32)

    c2 = _sc_kernel(tabs, l_edge_index, r_edge_index, mvec)

    bvec = jnp.stack([bl, br])
    out = _tail(c2.T, l_x, r_x, Wl, Wr, bvec, W1, b1.reshape(1, -1), W2, W3,
                b2.reshape(1, -1), b3.reshape(1, 1))
    return out.reshape(1)
